# baseline (speedup 1.0000x reference)
import functools

import jax
import jax.numpy as jnp
from jax import lax
from jax.experimental import pallas as pl
from jax.experimental.pallas import tpu as pltpu
from jax.experimental.pallas import tpu_sc as plsc

N = 10000
E = 320000
H = 128
NP = 10240
NSUB = 16
EPT = E // NSUB
CHP = 80
NCHP = EPT // CHP
BR = 256


def _sc_mesh():
    return plsc.VectorSubcoreMesh(core_axis_name="c", subcore_axis_name="s")


def _prep_body(idx2_hbm, w_hbm, out_hbm, deg_s, idx_big, w_big, o_big,
               idx_v, g_v, d_v, sem):
    cid = lax.axis_index("c")
    tid = lax.axis_index("s")
    ibase = cid * E + tid * EPT
    wbase = tid * EPT
    rows = NP // NSUB

    pltpu.sync_copy(idx2_hbm.at[pl.ds(ibase, EPT)], idx_big)
    pltpu.sync_copy(w_hbm.at[pl.ds(wbase, EPT)], w_big)

    z16 = jnp.zeros((16,), jnp.float32)
    for j in range(rows // 16):
        d_v[pl.ds(j * 16, 16)] = z16
    pltpu.sync_copy(d_v, deg_s.at[pl.ds(tid * rows, rows)])
    plsc.subcore_barrier()

    def scat(k, carry):
        off = k * CHP
        for j in range(CHP // 16):
            idx_v[pl.ds(j * 16, 16)] = idx_big[pl.ds(off + j * 16, 16)]
        pltpu.sync_copy(w_big.at[pl.ds(off, CHP)], deg_s.at[idx_v], add=True)
        return carry

    lax.fori_loop(0, NCHP, scat, 0)
    plsc.subcore_barrier()

    pltpu.sync_copy(deg_s.at[pl.ds(tid * rows, rows)], d_v)
    one = jnp.full((16,), 1.0, jnp.float32)
    for j in range(rows // 16):
        v = d_v[pl.ds(j * 16, 16)]
        d_v[pl.ds(j * 16, 16)] = jnp.where(v > 0.0, one / jnp.where(v > 0.0, v, one), z16)
    pltpu.sync_copy(d_v, deg_s.at[pl.ds(tid * rows, rows)])
    plsc.subcore_barrier()

    def coef(k, carry):
        off = k * CHP
        pltpu.async_copy(deg_s.at[idx_big.at[pl.ds(off, CHP)]], g_v, sem).wait()
        for j in range(CHP // 16):
            s = pl.ds(j * 16, 16)
            o_big[pl.ds(off + j * 16, 16)] = g_v[s] * w_big[pl.ds(off + j * 16, 16)]
        return carry

    lax.fori_loop(0, NCHP, coef, 0)
    pltpu.sync_copy(o_big, out_hbm.at[pl.ds(ibase, EPT)])


def _prep(idx2, w):
    return pl.kernel(
        _prep_body,
        out_type=jax.ShapeDtypeStruct((2 * E,), jnp.float32),
        mesh=_sc_mesh(),
        scratch_types=[
            pltpu.VMEM_SHARED((NP,), jnp.float32),
            pltpu.VMEM((EPT,), jnp.int32),
            pltpu.VMEM((EPT,), jnp.float32),
            pltpu.VMEM((EPT,), jnp.float32),
            pltpu.VMEM((CHP,), jnp.int32),
            pltpu.VMEM((CHP,), jnp.float32),
            pltpu.VMEM((NP // NSUB,), jnp.float32),
            pltpu.SemaphoreType.DMA,
        ],
    )(idx2, w)


def _spmv_body(F, xmode, two_pass, CH, x_hbm, ii_hbm, io_hbm, cf_hbm, out_hbm,
               acc, g0, g1, g2, g3,
               ii0, ii1, ii2, ii3, io0, io1, io2, io3, cf0, cf1, cf2, cf3,
               iot0, iot1, iot2, iot3, z_v,
               gm0, gm1, gm2, gm3, sm0, sm1, sm2, sm3, im0, im1, im2, im3):
    cid = lax.axis_index("c")
    tid = lax.axis_index("s")
    ept = EPT
    ebase = cid * E + tid * ept
    nchunk = ept // CH
    rows = NP // NSUB
    RB = 4
    gb = [g0, g1, g2, g3]
    gs = [gm0, gm1, gm2, gm3]
    ss = [sm0, sm1, sm2, sm3]
    iot = [iot0, iot1, iot2, iot3]
    iib = [ii0, ii1, ii2, ii3]
    iob = [io0, io1, io2, io3]
    cfb = [cf0, cf1, cf2, cf3]
    ism = [im0, im1, im2, im3]

    def idx_descs(blk, isl):
        off = ebase + blk * CH
        return [(ii_hbm.at[pl.ds(off, CH)], iib[isl]),
                (io_hbm.at[pl.ds(off, CH)], iob[isl]),
                (cf_hbm.at[pl.ds(off, CH)], cfb[isl])]

    def fire_idx(blk, isl):
        for src, dst in idx_descs(blk, isl):
            pltpu.async_copy(src, dst, ism[isl])

    def wait_idx(blk, isl):
        for src, dst in idx_descs(blk, isl):
            pltpu.make_async_copy(src, dst, ism[isl]).wait()

    def adj(isl, xoff):
        if xoff is not None:
            for j in range(CH // 16):
                s = pl.ds(j * 16, 16)
                iib[isl][s] = iib[isl][s] + xoff

    def fire_g(b):
        pltpu.async_copy(x_hbm.at[iib[b]], gb[b], gs[b])

    def wait_g(b):
        pltpu.make_async_copy(x_hbm.at[iib[b]], gb[b], gs[b]).wait()

    def wait_s(b):
        pltpu.make_async_copy(gb[b], acc.at[iot[b]], ss[b]).wait()

    def guard(pred, fn):
        if isinstance(pred, bool):
            if pred:
                fn()
        else:
            pl.when(pred)(fn)

    def do_chunk(k, j, xoff):
        b = j % RB
        b2 = (j + 2) % RB
        wait_g(b)

        @plsc.parallel_loop(0, CH // 16, 1, unroll=3)
        def scale(g):
            cv = cfb[b][pl.ds(g * 16, 16)]
            for l in range(16):
                c = cv[l]
                e = g * 16 + l
                for j3 in range(F // 16):
                    s = pl.ds(j3 * 16, 16)
                    gb[b][e, s] = gb[b][e, s] * c

        for j2 in range(CH // 16):
            s = pl.ds(j2 * 16, 16)
            iot[b][s] = iob[b][s]
        pltpu.async_copy(gb[b], acc.at[iot[b]], ss[b], add=True)

        def refill():
            guard(k >= 2, lambda: wait_s(b2))
            guard(k >= 2, lambda: wait_idx(k + 2, b2))
            adj(b2, xoff)
            fire_g(b2)

        guard(k + 2 < nchunk, refill)
        guard(k + RB < nchunk, lambda: fire_idx(k + RB, b))

    z16 = jnp.zeros((16,), jnp.float32)
    for r in range(16):
        for j in range(F // 16):
            z_v[r, pl.ds(j * 16, 16)] = z16

    def run_pass(p):
        if xmode == "none":
            xoff = None
        elif xmode == "cid":
            xoff = cid * NP
        elif xmode == "pass":
            xoff = None if p == 0 else p * NP
        else:
            xoff = cid * (2 * NP) + p * NP
        obase = (cid * (2 * NP) + p * NP) if two_pass else cid * NP

        for blk in range(RB):
            fire_idx(blk, blk)
        for blk in range(RB):
            wait_idx(blk, blk)
        for b in range(2):
            adj(b, xoff)
            fire_g(b)

        def zl(k, carry):
            pltpu.sync_copy(z_v, acc.at[pl.ds(tid * rows + k * 16, 16)])
            return carry

        lax.fori_loop(0, rows // 16, zl, 0)
        plsc.subcore_barrier()

        nmain = nchunk // RB

        def outer(i, carry):
            k0 = i * RB
            for j in range(RB):
                do_chunk(k0 + j, j, xoff)
            return carry

        lax.fori_loop(0, nmain, outer, 0)
        for k in range(nmain * RB, nchunk):
            do_chunk(k, k % RB, xoff)
        for b in range(RB):
            wait_s(b)

        plsc.subcore_barrier()
        pltpu.sync_copy(acc.at[pl.ds(tid * rows, rows)],
                        out_hbm.at[pl.ds(obase + tid * rows, rows)])

    for p in range(2 if two_pass else 1):
        run_pass(p)


def _spmv(F, xmode, two_pass, x, ii, io, cf):
    CH = 80
    oplanes = 4 if two_pass else 2
    return pl.kernel(
        functools.partial(_spmv_body, F, xmode, two_pass, CH),
        out_type=jax.ShapeDtypeStruct((oplanes * NP, F), jnp.float32),
        mesh=_sc_mesh(),
        scratch_types=(
            [pltpu.VMEM_SHARED((NP, F), jnp.float32)]
            + [pltpu.VMEM((CH, F), jnp.float32) for _ in range(4)]
            + [pltpu.VMEM((CH,), jnp.int32) for _ in range(8)]
            + [pltpu.VMEM((CH,), jnp.float32) for _ in range(4)]
            + [pltpu.VMEM((CH,), jnp.int32) for _ in range(4)]
            + [pltpu.VMEM((16, F), jnp.float32)]
            + [pltpu.SemaphoreType.DMA for _ in range(12)]
        ),
    )(x, ii, io, cf)


def _g1_body(s00, s01, s10, s11, s20, s21, s30, s31, s40, s41,
             w_ref, b_ref, g_ref, u_ref):
    a = jnp.concatenate(
        [s00[...], s01[...], s10[...], s11[...], s20[...], s21[...],
         s30[...], s31[...], s40[...], s41[...]], axis=1)
    v = lax.dot_general(a, w_ref[...], (((1,), (0,)), ((), ())),
                        precision=lax.Precision.HIGHEST)
    v = jax.nn.sigmoid(v + b_ref[0])
    r = v[:, :H]
    u = v[:, H:]
    g_ref[...] = r * s01[...]
    u_ref[...] = u


def _gates(x0, t12, t34, w1e, b1e):
    nb = NP // BR
    planes = [(x0, 0), (x0, 1), (t12, 0), (t12, 1), (t12, 2), (t12, 3),
              (t34, 0), (t34, 1), (t34, 2), (t34, 3)]
    specs = [pl.BlockSpec((BR, H), lambda i, q=q: (q * nb + i, 0))
             for _, q in planes]
    specs.append(pl.BlockSpec((1280, 2 * H), lambda i: (0, 0)))
    specs.append(pl.BlockSpec((8, 2 * H), lambda i: (0, 0)))
    args = [arr for arr, _ in planes] + [w1e, b1e]
    return pl.pallas_call(
        _g1_body,
        grid=(nb,),
        in_specs=specs,
        out_specs=[pl.BlockSpec((BR, H), lambda i: (i, 0)),
                   pl.BlockSpec((BR, H), lambda i: (i, 0))],
        out_shape=[jax.ShapeDtypeStruct((NP, H), jnp.float32),
                   jax.ShapeDtypeStruct((NP, H), jnp.float32)],
    )(*args)


def _g2_body(f0, f1, f2, f3, f4, g0, g1, g2, g3, g4,
             u_ref, hx_ref, w_ref, b_ref, o_ref):
    a = jnp.concatenate(
        [f0[...], f1[...], f2[...], f3[...], f4[...],
         g0[...], g1[...], g2[...], g3[...], g4[...]], axis=1)
    v = lax.dot_general(a, w_ref[...], (((1,), (0,)), ((), ())),
                        precision=lax.Precision.HIGHEST)
    c = jnp.tanh(v + b_ref[0])
    u = u_ref[...]
    o_ref[...] = u * hx_ref[...] + (1.0 - u) * c


def _final(x0, t12, t34, g, q12, q34, u, w2e, b2e):
    nb = NP // BR
    planes = [(x0, 0), (t12, 0), (t12, 2), (t34, 0), (t34, 2),
              (g, 0), (q12, 0), (q12, 1), (q34, 0), (q34, 1),
              (u, 0), (x0, 1)]
    specs = [pl.BlockSpec((BR, H), lambda i, q=q: (q * nb + i, 0))
             for _, q in planes]
    specs.append(pl.BlockSpec((1280, H), lambda i: (0, 0)))
    specs.append(pl.BlockSpec((8, H), lambda i: (0, 0)))
    args = [arr for arr, _ in planes] + [w2e, b2e]
    return pl.pallas_call(
        _g2_body,
        grid=(nb,),
        in_specs=specs,
        out_specs=pl.BlockSpec((BR, H), lambda i: (i, 0)),
        out_shape=jax.ShapeDtypeStruct((NP, H), jnp.float32),
    )(*args)


def _reorg_w1(W1):
    wr = W1.reshape(256, 5, 2 * H)
    terms = [wr[:, 0, :] - wr[:, 3, :] - wr[:, 4, :],
             wr[:, 1, :], wr[:, 2, :],
             2.0 * wr[:, 3, :], 2.0 * wr[:, 4, :]]
    return jnp.concatenate(terms, axis=0)


def _reorg_w2(W2):
    wr = W2.reshape(256, 5, H)
    terms = [wr[:, 0, :] - wr[:, 3, :] - wr[:, 4, :],
             wr[:, 1, :], wr[:, 2, :],
             2.0 * wr[:, 3, :], 2.0 * wr[:, 4, :]]
    fpart = jnp.concatenate([t[:H] for t in terms], axis=0)
    gpart = jnp.concatenate([t[H:] for t in terms], axis=0)
    return jnp.concatenate([fpart, gpart], axis=0)


def kernel(inputs, hx, edge_index, edge_weight, W1, b1, W2, b2):
    src = edge_index[0]
    dst = edge_index[1]
    idx2 = jnp.concatenate([src, dst])
    io2 = jnp.concatenate([dst, src])

    cc = _prep(idx2, edge_weight)

    x0 = jnp.zeros((2 * NP, H), jnp.float32)
    x0 = x0.at[:N].set(inputs).at[NP:NP + N].set(hx)

    t12 = _spmv(H, "pass", True, x0, idx2, io2, cc)
    t34 = _spmv(H, "cidpass", True, t12, idx2, io2, cc)

    w1e = _reorg_w1(W1)
    b1e = jnp.broadcast_to(b1[None, :], (8, 2 * H))
    g, u = _gates(x0, t12, t34, w1e, b1e)

    q12 = _spmv(H, "none", False, g, idx2, io2, cc)
    q34 = _spmv(H, "cid", False, q12, idx2, io2, cc)

    w2e = _reorg_w2(W2)
    b2e = jnp.broadcast_to(b2[None, :], (8, H))
    out = _final(x0, t12, t34, g, q12, q34, u, w2e, b2e)
    return out[:N]


# speedup vs baseline: 1.1098x; 1.1098x over previous
import functools

import jax
import jax.numpy as jnp
from jax import lax
from jax.experimental import pallas as pl
from jax.experimental.pallas import tpu as pltpu
from jax.experimental.pallas import tpu_sc as plsc

N = 10000
E = 320000
H = 128
NP = 10240
NSUB = 16
EPT = E // NSUB
CHP = 80
NCHP = EPT // CHP
BR = 256


def _sc_mesh():
    return plsc.VectorSubcoreMesh(core_axis_name="c", subcore_axis_name="s")


def _prep_body(idx2_hbm, w_hbm, out_hbm, deg_s, idx_big, w_big, o_big,
               idx_v, g_v, d_v, sem):
    cid = lax.axis_index("c")
    tid = lax.axis_index("s")
    ibase = cid * E + tid * EPT
    wbase = tid * EPT
    rows = NP // NSUB

    pltpu.sync_copy(idx2_hbm.at[pl.ds(ibase, EPT)], idx_big)
    pltpu.sync_copy(w_hbm.at[pl.ds(wbase, EPT)], w_big)

    z16 = jnp.zeros((16,), jnp.float32)
    for j in range(rows // 16):
        d_v[pl.ds(j * 16, 16)] = z16
    pltpu.sync_copy(d_v, deg_s.at[pl.ds(tid * rows, rows)])
    plsc.subcore_barrier()

    def scat(k, carry):
        off = k * CHP
        for j in range(CHP // 16):
            idx_v[pl.ds(j * 16, 16)] = idx_big[pl.ds(off + j * 16, 16)]
        pltpu.sync_copy(w_big.at[pl.ds(off, CHP)], deg_s.at[idx_v], add=True)
        return carry

    lax.fori_loop(0, NCHP, scat, 0)
    plsc.subcore_barrier()

    pltpu.sync_copy(deg_s.at[pl.ds(tid * rows, rows)], d_v)
    one = jnp.full((16,), 1.0, jnp.float32)
    for j in range(rows // 16):
        v = d_v[pl.ds(j * 16, 16)]
        d_v[pl.ds(j * 16, 16)] = jnp.where(v > 0.0, one / jnp.where(v > 0.0, v, one), z16)
    pltpu.sync_copy(d_v, deg_s.at[pl.ds(tid * rows, rows)])
    plsc.subcore_barrier()

    def coef(k, carry):
        off = k * CHP
        pltpu.async_copy(deg_s.at[idx_big.at[pl.ds(off, CHP)]], g_v, sem).wait()
        for j in range(CHP // 16):
            s = pl.ds(j * 16, 16)
            o_big[pl.ds(off + j * 16, 16)] = g_v[s] * w_big[pl.ds(off + j * 16, 16)]
        return carry

    lax.fori_loop(0, NCHP, coef, 0)
    pltpu.sync_copy(o_big, out_hbm.at[pl.ds(ibase, EPT)])


def _prep(idx2, w):
    return pl.kernel(
        _prep_body,
        out_type=jax.ShapeDtypeStruct((2 * E,), jnp.float32),
        mesh=_sc_mesh(),
        scratch_types=[
            pltpu.VMEM_SHARED((NP,), jnp.float32),
            pltpu.VMEM((EPT,), jnp.int32),
            pltpu.VMEM((EPT,), jnp.float32),
            pltpu.VMEM((EPT,), jnp.float32),
            pltpu.VMEM((CHP,), jnp.int32),
            pltpu.VMEM((CHP,), jnp.float32),
            pltpu.VMEM((NP // NSUB,), jnp.float32),
            pltpu.SemaphoreType.DMA,
        ],
    )(idx2, w)


def _spmv_body(F, xmode, two_pass, CH, x_hbm, ii_hbm, io_hbm, cf_hbm, out_hbm,
               acc, g0, g1, g2, g3,
               ii0, ii1, ii2, ii3, io0, io1, io2, io3, cf0, cf1, cf2, cf3,
               iot0, iot1, iot2, iot3, z_v,
               gm0, gm1, gm2, gm3, sm0, sm1, sm2, sm3, im0, im1, im2, im3):
    cid = lax.axis_index("c")
    tid = lax.axis_index("s")
    ept = EPT
    ebase = cid * E + tid * ept
    nchunk = ept // CH
    rows = NP // NSUB
    RB = 4
    gb = [g0, g1, g2, g3]
    gs = [gm0, gm1, gm2, gm3]
    ss = [sm0, sm1, sm2, sm3]
    iot = [iot0, iot1, iot2, iot3]
    iib = [ii0, ii1, ii2, ii3]
    iob = [io0, io1, io2, io3]
    cfb = [cf0, cf1, cf2, cf3]
    ism = [im0, im1, im2, im3]

    def idx_descs(blk, isl):
        off = ebase + blk * CH
        return [(ii_hbm.at[pl.ds(off, CH)], iib[isl]),
                (io_hbm.at[pl.ds(off, CH)], iob[isl]),
                (cf_hbm.at[pl.ds(off, CH)], cfb[isl])]

    def fire_idx(blk, isl):
        for src, dst in idx_descs(blk, isl):
            pltpu.async_copy(src, dst, ism[isl])

    def wait_idx(blk, isl):
        for src, dst in idx_descs(blk, isl):
            pltpu.make_async_copy(src, dst, ism[isl]).wait()

    def adj(isl, xoff):
        if xoff is not None:
            for j in range(CH // 16):
                s = pl.ds(j * 16, 16)
                iib[isl][s] = iib[isl][s] + xoff

    def fire_g(b):
        pltpu.async_copy(x_hbm.at[iib[b]], gb[b], gs[b])

    def wait_g(b):
        pltpu.make_async_copy(x_hbm.at[iib[b]], gb[b], gs[b]).wait()

    def wait_s(b):
        pltpu.make_async_copy(gb[b], acc.at[iot[b]], ss[b]).wait()

    def guard(pred, fn):
        if isinstance(pred, bool):
            if pred:
                fn()
        else:
            pl.when(pred)(fn)

    def do_chunk(k, j, xoff):
        b = j % RB
        b2 = (j + 2) % RB
        wait_g(b)

        @plsc.parallel_loop(0, CH // 16, 1, unroll=2)
        def scale(g):
            cv = cfb[b][pl.ds(g * 16, 16)]
            for l in range(16):
                c = cv[l]
                e = g * 16 + l
                for j3 in range(F // 16):
                    s = pl.ds(j3 * 16, 16)
                    gb[b][e, s] = gb[b][e, s] * c

        for j2 in range(CH // 16):
            s = pl.ds(j2 * 16, 16)
            iot[b][s] = iob[b][s]
        pltpu.async_copy(gb[b], acc.at[iot[b]], ss[b], add=True)

        def refill():
            guard(k >= 2, lambda: wait_s(b2))
            guard(k >= 2, lambda: wait_idx(k + 2, b2))
            adj(b2, xoff)
            fire_g(b2)

        guard(k + 2 < nchunk, refill)
        guard(k + RB < nchunk, lambda: fire_idx(k + RB, b))

    z16 = jnp.zeros((16,), jnp.float32)
    for r in range(16):
        for j in range(F // 16):
            z_v[r, pl.ds(j * 16, 16)] = z16

    def run_pass(p):
        if xmode == "none":
            xoff = None
        elif xmode == "cid":
            xoff = cid * NP
        elif xmode == "pass":
            xoff = None if p == 0 else p * NP
        else:
            xoff = cid * (2 * NP) + p * NP
        obase = (cid * (2 * NP) + p * NP) if two_pass else cid * NP

        for blk in range(RB):
            fire_idx(blk, blk)
        for blk in range(RB):
            wait_idx(blk, blk)
        for b in range(2):
            adj(b, xoff)
            fire_g(b)

        def zl(k, carry):
            pltpu.sync_copy(z_v, acc.at[pl.ds(tid * rows + k * 16, 16)])
            return carry

        lax.fori_loop(0, rows // 16, zl, 0)
        plsc.subcore_barrier()

        nmain = nchunk // RB

        def outer(i, carry):
            k0 = i * RB
            for j in range(RB):
                do_chunk(k0 + j, j, xoff)
            return carry

        lax.fori_loop(0, nmain, outer, 0)
        for k in range(nmain * RB, nchunk):
            do_chunk(k, k % RB, xoff)
        for b in range(RB):
            wait_s(b)

        plsc.subcore_barrier()
        pltpu.sync_copy(acc.at[pl.ds(tid * rows, rows)],
                        out_hbm.at[pl.ds(obase + tid * rows, rows)])

    for p in range(2 if two_pass else 1):
        run_pass(p)


def _spmv(F, xmode, two_pass, x, ii, io, cf):
    CH = 80
    oplanes = 4 if two_pass else 2
    return pl.kernel(
        functools.partial(_spmv_body, F, xmode, two_pass, CH),
        out_type=jax.ShapeDtypeStruct((oplanes * NP, F), jnp.float32),
        mesh=_sc_mesh(),
        scratch_types=(
            [pltpu.VMEM_SHARED((NP, F), jnp.float32)]
            + [pltpu.VMEM((CH, F), jnp.float32) for _ in range(4)]
            + [pltpu.VMEM((CH,), jnp.int32) for _ in range(8)]
            + [pltpu.VMEM((CH,), jnp.float32) for _ in range(4)]
            + [pltpu.VMEM((CH,), jnp.int32) for _ in range(4)]
            + [pltpu.VMEM((16, F), jnp.float32)]
            + [pltpu.SemaphoreType.DMA for _ in range(12)]
        ),
    )(x, ii, io, cf)


def _g1_body(s00, s01, s10, s11, s20, s21, s30, s31, s40, s41,
             w_ref, b_ref, g_ref, u_ref):
    a = jnp.concatenate(
        [s00[...], s01[...], s10[...], s11[...], s20[...], s21[...],
         s30[...], s31[...], s40[...], s41[...]], axis=1)
    v = lax.dot_general(a, w_ref[...], (((1,), (0,)), ((), ())),
                        precision=lax.Precision.HIGHEST)
    v = jax.nn.sigmoid(v + b_ref[0])
    r = v[:, :H]
    u = v[:, H:]
    g_ref[...] = r * s01[...]
    u_ref[...] = u


def _gates(x0, t12, t34, w1e, b1e):
    nb = NP // BR
    planes = [(x0, 0), (x0, 1), (t12, 0), (t12, 1), (t12, 2), (t12, 3),
              (t34, 0), (t34, 1), (t34, 2), (t34, 3)]
    specs = [pl.BlockSpec((BR, H), lambda i, q=q: (q * nb + i, 0))
             for _, q in planes]
    specs.append(pl.BlockSpec((1280, 2 * H), lambda i: (0, 0)))
    specs.append(pl.BlockSpec((8, 2 * H), lambda i: (0, 0)))
    args = [arr for arr, _ in planes] + [w1e, b1e]
    return pl.pallas_call(
        _g1_body,
        grid=(nb,),
        in_specs=specs,
        out_specs=[pl.BlockSpec((BR, H), lambda i: (i, 0)),
                   pl.BlockSpec((BR, H), lambda i: (i, 0))],
        out_shape=[jax.ShapeDtypeStruct((NP, H), jnp.float32),
                   jax.ShapeDtypeStruct((NP, H), jnp.float32)],
    )(*args)


def _g2_body(f0, f1, f2, f3, f4, g0, g1, g2, g3, g4,
             u_ref, hx_ref, w_ref, b_ref, o_ref):
    a = jnp.concatenate(
        [f0[...], f1[...], f2[...], f3[...], f4[...],
         g0[...], g1[...], g2[...], g3[...], g4[...]], axis=1)
    v = lax.dot_general(a, w_ref[...], (((1,), (0,)), ((), ())),
                        precision=lax.Precision.HIGHEST)
    c = jnp.tanh(v + b_ref[0])
    u = u_ref[...]
    o_ref[...] = u * hx_ref[...] + (1.0 - u) * c


def _final(x0, t12, t34, g, q12, q34, u, w2e, b2e):
    nb = NP // BR
    planes = [(x0, 0), (t12, 0), (t12, 2), (t34, 0), (t34, 2),
              (g, 0), (q12, 0), (q12, 1), (q34, 0), (q34, 1),
              (u, 0), (x0, 1)]
    specs = [pl.BlockSpec((BR, H), lambda i, q=q: (q * nb + i, 0))
             for _, q in planes]
    specs.append(pl.BlockSpec((1280, H), lambda i: (0, 0)))
    specs.append(pl.BlockSpec((8, H), lambda i: (0, 0)))
    args = [arr for arr, _ in planes] + [w2e, b2e]
    return pl.pallas_call(
        _g2_body,
        grid=(nb,),
        in_specs=specs,
        out_specs=pl.BlockSpec((BR, H), lambda i: (i, 0)),
        out_shape=jax.ShapeDtypeStruct((NP, H), jnp.float32),
    )(*args)


def _reorg_w1(W1):
    wr = W1.reshape(256, 5, 2 * H)
    terms = [wr[:, 0, :] - wr[:, 3, :] - wr[:, 4, :],
             wr[:, 1, :], wr[:, 2, :],
             2.0 * wr[:, 3, :], 2.0 * wr[:, 4, :]]
    return jnp.concatenate(terms, axis=0)


def _reorg_w2(W2):
    wr = W2.reshape(256, 5, H)
    terms = [wr[:, 0, :] - wr[:, 3, :] - wr[:, 4, :],
             wr[:, 1, :], wr[:, 2, :],
             2.0 * wr[:, 3, :], 2.0 * wr[:, 4, :]]
    fpart = jnp.concatenate([t[:H] for t in terms], axis=0)
    gpart = jnp.concatenate([t[H:] for t in terms], axis=0)
    return jnp.concatenate([fpart, gpart], axis=0)


def kernel(inputs, hx, edge_index, edge_weight, W1, b1, W2, b2):
    src = edge_index[0]
    dst = edge_index[1]
    idx2 = jnp.concatenate([src, dst])
    io2 = jnp.concatenate([dst, src])

    cc = _prep(idx2, edge_weight)

    x0 = jnp.zeros((2 * NP, H), jnp.float32)
    x0 = x0.at[:N].set(inputs).at[NP:NP + N].set(hx)

    t12 = _spmv(H, "pass", True, x0, idx2, io2, cc)
    t34 = _spmv(H, "cidpass", True, t12, idx2, io2, cc)

    w1e = _reorg_w1(W1)
    b1e = jnp.broadcast_to(b1[None, :], (8, 2 * H))
    g, u = _gates(x0, t12, t34, w1e, b1e)

    q12 = _spmv(H, "none", False, g, idx2, io2, cc)
    q34 = _spmv(H, "cid", False, q12, idx2, io2, cc)

    w2e = _reorg_w2(W2)
    b2e = jnp.broadcast_to(b2[None, :], (8, H))
    out = _final(x0, t12, t34, g, q12, q34, u, w2e, b2e)
    return out[:N]
